# baseline (speedup 1.0000x reference)
import jax
import jax.numpy as jnp
from jax import lax
from jax.experimental import pallas as pl
from jax.experimental.pallas import tpu as pltpu
from jax.experimental.pallas import tpu_sc as plsc

N = 4096
D = 2048
B = 256

NC = 2
NS = 16
NW = NC * NS
RPW = N // NW
RC = 8
NCHUNK = RPW // RC


def _tc_kernel(ub_ref, lb_ref, Wu_in_ref, bu_in_ref, Wl_in_ref, bl_in_ref,
               alpha_ref,
               new_ub_ref, new_lb_ref, Wu_ref, bu_ref, Wl_ref, bl_ref,
               Wu2_ref, bu2_ref, bl2_ref):
    i = pl.program_id(0)
    ub = ub_ref[:]
    lb = lb_ref[:]
    alpha = alpha_ref[:]
    bu_in = bu_in_ref[:]
    bl_in = bl_in_ref[:]

    neg = ub <= 0.0
    pos = lb >= 0.0
    cross = jnp.logical_not(jnp.logical_or(neg, pos))
    alpha_c = jnp.clip(alpha, 0.0, 1.0)
    denom = jnp.where(cross, ub - lb, 1.0)
    a = jnp.where(cross, ub / denom, 0.0)
    b = -lb * a

    new_ub_ref[:] = jnp.where(neg, 0.0, ub)
    new_lb_ref[:] = jnp.where(pos, lb, jnp.where(cross, alpha_c * lb, 0.0))
    bu_ref[:] = jnp.where(pos, bu_in, jnp.where(cross, bu_in + b, 0.0))
    bl_ref[:] = jnp.where(pos, bl_in, jnp.where(cross, bu_in, 0.0))
    bu2_ref[:] = jnp.where(cross, b, 0.0)
    bl2_ref[:] = jnp.zeros_like(b)

    u_scale = jnp.where(pos, 1.0, a)
    l_scale = jnp.where(pos, 1.0, jnp.where(cross, alpha_c, 0.0))
    Wu_ref[:, :] = u_scale[:, None] * Wu_in_ref[:, :]
    Wl_ref[:, :] = l_scale[:, None] * Wl_in_ref[:, :]

    du = jnp.where(cross, a, 1.0)
    rows = lax.broadcasted_iota(jnp.int32, (B, N), 0) + i * B
    cols = lax.broadcasted_iota(jnp.int32, (B, N), 1)
    Wu2_ref[:, :] = jnp.where(rows == cols, du[:, None], 0.0)


def _tc_part(ub, lb, W_upper, b_upper, W_lower, b_lower, alpha):
    grid = (N // B,)
    vec_spec = pl.BlockSpec((B,), lambda i: (i,))
    mat_spec = pl.BlockSpec((B, D), lambda i: (i, 0))
    diag_spec = pl.BlockSpec((B, N), lambda i: (i, 0))
    f32 = jnp.float32
    out_shapes = (
        jax.ShapeDtypeStruct((N,), f32),
        jax.ShapeDtypeStruct((N,), f32),
        jax.ShapeDtypeStruct((N, D), f32),
        jax.ShapeDtypeStruct((N,), f32),
        jax.ShapeDtypeStruct((N, D), f32),
        jax.ShapeDtypeStruct((N,), f32),
        jax.ShapeDtypeStruct((N, N), f32),
        jax.ShapeDtypeStruct((N,), f32),
        jax.ShapeDtypeStruct((N,), f32),
    )
    out_specs = (vec_spec, vec_spec, mat_spec, vec_spec, mat_spec, vec_spec,
                 diag_spec, vec_spec, vec_spec)
    in_specs = (vec_spec, vec_spec, mat_spec, vec_spec, mat_spec, vec_spec,
                vec_spec)
    return pl.pallas_call(
        _tc_kernel,
        grid=grid,
        in_specs=in_specs,
        out_specs=out_specs,
        out_shape=out_shapes,
    )(ub, lb, W_upper, b_upper, W_lower, b_lower, alpha)


def _sc_body(wl2_hbm, zb, sem):
    wid = lax.axis_index("s") * NC + lax.axis_index("c")
    row0 = wid * RPW

    zeros16 = jnp.zeros((16,), jnp.float32)
    for r in range(RC):
        def _zero(k, _, r=r):
            zb[r, pl.ds(k * 16, 16)] = zeros16
            return 0
        lax.fori_loop(0, N // 16, _zero, 0)

    copies = []
    for c in range(NCHUNK):
        copies.append(pltpu.async_copy(
            zb, wl2_hbm.at[pl.ds(row0 + c * RC, RC), :], sem))
    for cp in copies:
        cp.wait()


def _sc_zero_fill():
    mesh = plsc.VectorSubcoreMesh(core_axis_name="c", subcore_axis_name="s")
    fill_kernel = pl.kernel(
        _sc_body,
        out_type=jax.ShapeDtypeStruct((N, N), jnp.float32),
        mesh=mesh,
        compiler_params=pltpu.CompilerParams(needs_layout_passes=False),
        scratch_types=(
            pltpu.VMEM((RC, N), jnp.float32),
            pltpu.SemaphoreType.DMA,
        ),
    )
    return fill_kernel()


def _tc_patch_kernel(ub_ref, lb_ref, alpha_ref, wl2_in_ref, wl2_ref):
    ub = ub_ref[:]
    lb = lb_ref[:]
    alpha = alpha_ref[:]
    cross = jnp.logical_and(ub > 0.0, lb < 0.0)
    alpha_c = jnp.clip(alpha, 0.0, 1.0)
    dl = jnp.where(cross, alpha_c, 1.0)
    r = lax.broadcasted_iota(jnp.int32, (RPW, RPW), 0)
    c = lax.broadcasted_iota(jnp.int32, (RPW, RPW), 1)
    wl2_ref[:, :] = jnp.where(r == c, dl[:, None], 0.0)


def _tc_diag_patch(ub, lb, alpha, wl2_zeros):
    vec_spec = pl.BlockSpec((RPW,), lambda i: (i,))
    return pl.pallas_call(
        _tc_patch_kernel,
        grid=(NW,),
        in_specs=(vec_spec, vec_spec, vec_spec,
                  pl.BlockSpec(memory_space=pl.ANY)),
        out_specs=pl.BlockSpec((RPW, RPW), lambda i: (i, i)),
        out_shape=jax.ShapeDtypeStruct((N, N), jnp.float32),
        input_output_aliases={3: 0},
    )(ub, lb, alpha, wl2_zeros)


@jax.jit
def kernel(ub, lb, W_upper, b_upper, W_lower, b_lower, alpha, input_ub, input_lb):
    del input_ub, input_lb
    wl2_zeros = _sc_zero_fill()
    (new_ub, new_lb, Wu, bu, Wl, bl, Wu2, bu2, bl2) = _tc_part(
        ub, lb, W_upper, b_upper, W_lower, b_lower, alpha)
    wl2 = _tc_diag_patch(ub, lb, alpha, wl2_zeros)
    return (new_ub, new_lb, Wu, bu, Wl, bl, Wu2, bu2, wl2, bl2)


# speedup vs baseline: 1.4158x; 1.4158x over previous
import jax
import jax.numpy as jnp
from jax import lax
from jax.experimental import pallas as pl

N = 4096
D = 2048
B = 256


def _relu_kernel(ub_ref, lb_ref, Wu_in_ref, bu_in_ref, Wl_in_ref, bl_in_ref,
                 alpha_ref,
                 new_ub_ref, new_lb_ref, Wu_ref, bu_ref, Wl_ref, bl_ref,
                 Wu2_ref, bu2_ref, Wl2_ref, bl2_ref):
    i = pl.program_id(0)
    ub = ub_ref[:]
    lb = lb_ref[:]
    alpha = alpha_ref[:]
    bu_in = bu_in_ref[:]
    bl_in = bl_in_ref[:]

    neg = ub <= 0.0
    pos = lb >= 0.0
    cross = jnp.logical_not(jnp.logical_or(neg, pos))
    alpha_c = jnp.clip(alpha, 0.0, 1.0)
    denom = jnp.where(cross, ub - lb, 1.0)
    a = jnp.where(cross, ub / denom, 0.0)
    b = -lb * a

    new_ub_ref[:] = jnp.where(neg, 0.0, ub)
    new_lb_ref[:] = jnp.where(pos, lb, jnp.where(cross, alpha_c * lb, 0.0))
    bu_ref[:] = jnp.where(pos, bu_in, jnp.where(cross, bu_in + b, 0.0))
    bl_ref[:] = jnp.where(pos, bl_in, jnp.where(cross, bu_in, 0.0))
    bu2_ref[:] = jnp.where(cross, b, 0.0)
    bl2_ref[:] = jnp.zeros_like(b)

    u_scale = jnp.where(pos, 1.0, a)
    l_scale = jnp.where(pos, 1.0, jnp.where(cross, alpha_c, 0.0))
    Wu_ref[:, :] = u_scale[:, None] * Wu_in_ref[:, :]
    Wl_ref[:, :] = l_scale[:, None] * Wl_in_ref[:, :]

    du = jnp.where(cross, a, 1.0)
    dl = jnp.where(cross, alpha_c, 1.0)
    rows = lax.broadcasted_iota(jnp.int32, (B, N), 0) + i * B
    cols = lax.broadcasted_iota(jnp.int32, (B, N), 1)
    on_diag = rows == cols
    Wu2_ref[:, :] = jnp.where(on_diag, du[:, None], 0.0)
    Wl2_ref[:, :] = jnp.where(on_diag, dl[:, None], 0.0)


@jax.jit
def kernel(ub, lb, W_upper, b_upper, W_lower, b_lower, alpha, input_ub, input_lb):
    del input_ub, input_lb
    grid = (N // B,)
    vec_spec = pl.BlockSpec((B,), lambda i: (i,))
    mat_spec = pl.BlockSpec((B, D), lambda i: (i, 0))
    diag_spec = pl.BlockSpec((B, N), lambda i: (i, 0))
    f32 = jnp.float32
    out_shapes = (
        jax.ShapeDtypeStruct((N,), f32),
        jax.ShapeDtypeStruct((N,), f32),
        jax.ShapeDtypeStruct((N, D), f32),
        jax.ShapeDtypeStruct((N,), f32),
        jax.ShapeDtypeStruct((N, D), f32),
        jax.ShapeDtypeStruct((N,), f32),
        jax.ShapeDtypeStruct((N, N), f32),
        jax.ShapeDtypeStruct((N,), f32),
        jax.ShapeDtypeStruct((N, N), f32),
        jax.ShapeDtypeStruct((N,), f32),
    )
    out_specs = (vec_spec, vec_spec, mat_spec, vec_spec, mat_spec, vec_spec,
                 diag_spec, vec_spec, diag_spec, vec_spec)
    in_specs = (vec_spec, vec_spec, mat_spec, vec_spec, mat_spec, vec_spec,
                vec_spec)
    return pl.pallas_call(
        _relu_kernel,
        grid=grid,
        in_specs=in_specs,
        out_specs=out_specs,
        out_shape=out_shapes,
    )(ub, lb, W_upper, b_upper, W_lower, b_lower, alpha)
